# Initial kernel scaffold; baseline (speedup 1.0000x reference)
import jax
import jax.numpy as jnp
from jax.experimental import pallas as pl


def kernel(x, input_geom, latent_queries, Wp, bp, W0, b0, W1, b1, W2, b2, W3, b3):
    raise NotImplementedError("write your pallas kernel here")


# speedup vs baseline: 5.3878x; 5.3878x over previous
import functools
import jax
import jax.numpy as jnp
from jax.experimental import pallas as pl
from jax.experimental.pallas import tpu as pltpu

RADIUS = 0.08
N_PAD = 10240
QB = 16
NC = 2048


def _proj_body(x_ref, Wp_ref, bp_ref, f_ref):
    f_ref[:] = jnp.dot(x_ref[:], Wp_ref[:],
                       preferred_element_type=jnp.float32) + bp_ref[:]


def _gelu(v):
    return 0.5 * v * (1.0 + jax.lax.erf(v * 0.7071067811865476))


def _dense_body(lq_ref, ig_ref, f_ref, W0q_ref, W0g_ref, b0_ref, W1_ref,
                b1_ref, W2_ref, b2_ref, W3_ref, b3_ref, out_ref,
                acc_ref, cnt_ref):
    j = pl.program_id(1)
    nj = pl.num_programs(1)

    @pl.when(j == 0)
    def _():
        acc_ref[:] = jnp.zeros_like(acc_ref)
        cnt_ref[:] = jnp.zeros_like(cnt_ref)

    ig = ig_ref[:]
    lq = lq_ref[:]
    f = f_ref[:]
    igW = jnp.dot(ig, W0g_ref[:], preferred_element_type=jnp.float32)
    ig2 = jnp.sum(ig * ig, axis=1, keepdims=True)
    igb = ig.astype(jnp.bfloat16)

    r2 = jnp.float32(RADIUS * RADIUS)
    for i in range(QB):
        lqi = lq[i:i + 1, :]
        qrow = jnp.dot(lqi, W0q_ref[:],
                       preferred_element_type=jnp.float32) + b0_ref[:]
        cross = jnp.dot(igb, lqi.astype(jnp.bfloat16).T,
                        preferred_element_type=jnp.float32)
        d2 = jnp.sum(lqi * lqi) + ig2 - 2.0 * cross
        h = _gelu(igW + qrow)
        h = _gelu(jnp.dot(h, W1_ref[:], preferred_element_type=jnp.float32)
                  + b1_ref[:])
        h = _gelu(jnp.dot(h, W2_ref[:], preferred_element_type=jnp.float32)
                  + b2_ref[:])
        k = jnp.dot(h, W3_ref[:], preferred_element_type=jnp.float32) \
            + b3_ref[:]
        msk = d2 < r2
        contrib = jnp.where(msk, k * f, 0.0)
        acc_ref[i:i + 1, :] += jnp.sum(contrib, axis=0, keepdims=True)
        cnt_ref[i:i + 1, :] += jnp.sum(msk.astype(jnp.float32))

    @pl.when(j == nj - 1)
    def _():
        out_ref[:] = acc_ref[:] / jnp.maximum(cnt_ref[:], 1.0)


def kernel(x, input_geom, latent_queries, Wp, bp, W0, b0, W1, b1, W2, b2,
           W3, b3):
    ig = input_geom[0]
    n = ig.shape[0]
    lq = latent_queries[0]
    grid_shape = lq.shape[:-1]
    lqf = lq.reshape(-1, 3)
    q = lqf.shape[0]

    xp = jnp.zeros((N_PAD, x.shape[-1]), jnp.float32).at[:n].set(x[0])
    igp = jnp.full((N_PAD, 3), 1e3, jnp.float32).at[:n].set(ig)

    f = pl.pallas_call(
        _proj_body,
        grid=(N_PAD // 2048,),
        in_specs=[
            pl.BlockSpec((2048, x.shape[-1]), lambda i: (i, 0)),
            pl.BlockSpec((x.shape[-1], Wp.shape[-1]), lambda i: (0, 0)),
            pl.BlockSpec((1, Wp.shape[-1]), lambda i: (0, 0)),
        ],
        out_specs=pl.BlockSpec((2048, Wp.shape[-1]), lambda i: (i, 0)),
        out_shape=jax.ShapeDtypeStruct((N_PAD, Wp.shape[-1]), jnp.float32),
    )(xp, Wp, bp.reshape(1, -1))

    P = Wp.shape[-1]
    H = W0.shape[-1]
    out = pl.pallas_call(
        _dense_body,
        grid=(q // QB, N_PAD // NC),
        in_specs=[
            pl.BlockSpec((QB, 3), lambda i, j: (i, 0)),
            pl.BlockSpec((NC, 3), lambda i, j: (j, 0)),
            pl.BlockSpec((NC, P), lambda i, j: (j, 0)),
            pl.BlockSpec((3, H), lambda i, j: (0, 0)),
            pl.BlockSpec((3, H), lambda i, j: (0, 0)),
            pl.BlockSpec((1, H), lambda i, j: (0, 0)),
            pl.BlockSpec((H, H), lambda i, j: (0, 0)),
            pl.BlockSpec((1, H), lambda i, j: (0, 0)),
            pl.BlockSpec((H, H), lambda i, j: (0, 0)),
            pl.BlockSpec((1, H), lambda i, j: (0, 0)),
            pl.BlockSpec((H, P), lambda i, j: (0, 0)),
            pl.BlockSpec((1, P), lambda i, j: (0, 0)),
        ],
        out_specs=pl.BlockSpec((QB, P), lambda i, j: (i, 0)),
        out_shape=jax.ShapeDtypeStruct((q, P), jnp.float32),
        scratch_shapes=[
            pltpu.VMEM((QB, P), jnp.float32),
            pltpu.VMEM((QB, P), jnp.float32),
        ],
    )(lqf, igp, f, W0[:3], W0[3:], b0.reshape(1, -1), W1, b1.reshape(1, -1),
      W2, b2.reshape(1, -1), W3, b3.reshape(1, -1))

    return out.reshape((1,) + grid_shape + (P,))


# speedup vs baseline: 136.9785x; 25.4237x over previous
import functools
import jax
import jax.numpy as jnp
from jax import lax
from jax.experimental import pallas as pl
from jax.experimental.pallas import tpu as pltpu
from jax.experimental.pallas import tpu_sc as plsc

RADIUS = 0.08
N_PAD = 10240
NW = N_PAD // 32
Q = 4096
QB = 128
NSC = 32
QPW = Q // NSC
CAP = 4096
E = NSC * CAP
PROJ = 256
MLP_BLK = 1024


def _proj_body(x_ref, Wp_ref, bp_ref, f_ref):
    f_ref[:] = jnp.dot(x_ref[:], Wp_ref[:],
                       preferred_element_type=jnp.float32) + bp_ref[:]


def _mask_body(lq_ref, igT_ref, lq2_ref, ig2_ref, plo_ref, phi_ref, w_ref):
    lqb = lq_ref[:].astype(jnp.bfloat16)
    igTb = igT_ref[:].astype(jnp.bfloat16)
    cross = jnp.dot(lqb, igTb, preferred_element_type=jnp.float32)
    d2 = (lq2_ref[:] + ig2_ref[:]) - 2.0 * cross
    mv = (d2 < jnp.float32(RADIUS * RADIUS)).astype(jnp.float32)
    lo = jnp.dot(mv, plo_ref[:],
                 preferred_element_type=jnp.float32).astype(jnp.int32)
    hi = jnp.dot(mv, phi_ref[:],
                 preferred_element_type=jnp.float32).astype(jnp.int32)
    w_ref[:] = lo | (hi << 16)


def _search_body(igx, igy, igz, lqx, lqy, lqz, words,
                 kin_hbm, en_hbm, eq_hbm, cnt_hbm, start_hbm,
                 igx_v, igy_v, igz_v, lqx_v, lqy_v, lqz_v,
                 words_v, aw_v, kin_v, en_v, eq_v, cnt_v, start_v):
    wid = lax.axis_index("s") * 2 + lax.axis_index("c")
    qbase = wid * QPW

    pltpu.sync_copy(igx, igx_v)
    pltpu.sync_copy(igy, igy_v)
    pltpu.sync_copy(igz, igz_v)
    pltpu.sync_copy(lqx.at[pl.ds(qbase, QPW)], lqx_v)
    pltpu.sync_copy(lqy.at[pl.ds(qbase, QPW)], lqy_v)
    pltpu.sync_copy(lqz.at[pl.ds(qbase, QPW)], lqz_v)
    pltpu.sync_copy(words.at[pl.ds(wid * QPW * NW, QPW * NW)], words_v)

    zi = jnp.zeros((16,), jnp.int32)
    zf = jnp.zeros((16,), jnp.float32)

    def _zero(i, _):
        en_v[pl.ds(i * 16, 16)] = zi
        eq_v[pl.ds(i * 16, 16)] = zi
        return 0

    lax.fori_loop(0, CAP // 16, _zero, 0)

    def _zerok(i, _):
        kin_v[pl.ds(i * 16, 16)] = zf
        return 0

    lax.fori_loop(0, CAP * 8 // 16, _zerok, 0)

    iota = lax.iota(jnp.int32, 16)

    def _query(qloc, carry):
        pos, cvec, svec = carry
        lane = qloc % 16
        svec = jnp.where(iota == lane, jnp.full((16,), pos, jnp.int32),
                         svec)
        qidx = jnp.full((16,), qloc, jnp.int32)
        qx = plsc.load_gather(lqx_v, [qidx])
        qy = plsc.load_gather(lqy_v, [qidx])
        qz = plsc.load_gather(lqz_v, [qidx])
        wbase = qloc * NW

        def _cw(t, na):
            wvec = words_v[pl.ds(wbase + t * 16, 16)]
            m = wvec != 0
            mi = m.astype(jnp.int32)
            rows = na + plsc.cumsum(mi) - 1
            plsc.store_scatter(aw_v, [rows], t * 16 + iota, mask=m)
            return na + jnp.sum(mi)

        na = lax.fori_loop(0, NW // 16, _cw, jnp.int32(0))

        def _aw(e, pos):
            tsp = plsc.load_gather(aw_v, [jnp.full((16,), e, jnp.int32)])
            wval = plsc.load_gather(words_v, [tsp + wbase])
            for h in range(2):
                bits = (wval >> (iota + 16 * h)) & 1
                m = bits == 1
                pidx = tsp * 32 + (16 * h) + iota
                mi = m.astype(jnp.int32)
                npos = jnp.minimum(pos, CAP - 16)
                rows = npos + plsc.cumsum(mi) - 1
                xs = plsc.load_gather(igx_v, [pidx])
                ys = plsc.load_gather(igy_v, [pidx])
                zs = plsc.load_gather(igz_v, [pidx])
                r8 = rows * 8
                plsc.store_scatter(kin_v, [r8], qx, mask=m)
                plsc.store_scatter(kin_v, [r8 + 1], qy, mask=m)
                plsc.store_scatter(kin_v, [r8 + 2], qz, mask=m)
                plsc.store_scatter(kin_v, [r8 + 3], xs, mask=m)
                plsc.store_scatter(kin_v, [r8 + 4], ys, mask=m)
                plsc.store_scatter(kin_v, [r8 + 5], zs, mask=m)
                plsc.store_scatter(en_v, [rows], pidx, mask=m)
                plsc.store_scatter(eq_v, [rows], qidx, mask=m)
                pos = pos + jnp.sum(mi)
            return pos

        pos2 = lax.fori_loop(0, na, _aw, pos)
        cvec = jnp.where(iota == lane, jnp.full((16,), pos2 - pos,
                                                jnp.int32), cvec)

        @pl.when(lane == 15)
        def _():
            cnt_v[pl.ds(qloc - 15, 16)] = cvec
            start_v[pl.ds(qloc - 15, 16)] = svec

        return (pos2, cvec, svec)

    zi16 = jnp.zeros((16,), jnp.int32)
    lax.fori_loop(0, QPW, _query, (jnp.int32(0), zi16, zi16))

    pltpu.sync_copy(kin_v, kin_hbm.at[pl.ds(wid * CAP * 8, CAP * 8)])
    pltpu.sync_copy(en_v, en_hbm.at[pl.ds(wid * CAP, CAP)])
    pltpu.sync_copy(eq_v, eq_hbm.at[pl.ds(wid * CAP, CAP)])
    pltpu.sync_copy(cnt_v, cnt_hbm.at[pl.ds(qbase, QPW)])
    pltpu.sync_copy(start_v, start_hbm.at[pl.ds(qbase, QPW)])


def _gelu(v):
    return 0.5 * v * (1.0 + jax.lax.erf(v * 0.7071067811865476))


def _mlp_body(kin_ref, W0_ref, b0_ref, W1_ref, b1_ref, W2_ref, b2_ref,
              W3_ref, b3_ref, k_ref):
    h = _gelu(jnp.dot(kin_ref[:], W0_ref[:],
                      preferred_element_type=jnp.float32) + b0_ref[:])
    h = _gelu(jnp.dot(h, W1_ref[:], preferred_element_type=jnp.float32)
              + b1_ref[:])
    h = _gelu(jnp.dot(h, W2_ref[:], preferred_element_type=jnp.float32)
              + b2_ref[:])
    k_ref[:] = jnp.dot(h, W3_ref[:], preferred_element_type=jnp.float32) \
        + b3_ref[:]


def _agg_body(kflat, f2d, en_hbm, eq_hbm, cnt_hbm, start_hbm, o_hbm,
              en_v, eq_v, cnt_v, start_v, kbuf, fbuf, acc, ksem, fsem):
    wid = lax.axis_index("s") * 2 + lax.axis_index("c")
    qbase = wid * QPW

    pltpu.sync_copy(en_hbm.at[pl.ds(wid * CAP, CAP)], en_v)
    pltpu.sync_copy(eq_hbm.at[pl.ds(wid * CAP, CAP)],
                    eq_v.at[pl.ds(0, CAP)])
    pltpu.sync_copy(cnt_hbm.at[pl.ds(qbase, QPW)], cnt_v.at[pl.ds(0, QPW)])
    pltpu.sync_copy(start_hbm.at[pl.ds(qbase, QPW)],
                    start_v.at[pl.ds(0, QPW)])

    zf = jnp.zeros((16,), jnp.float32)

    def _zero(i, _):
        acc[pl.ds(i * 16, 16)] = zf
        return 0

    lax.fori_loop(0, QPW * PROJ // 16, _zero, 0)

    ne = start_v[pl.ds(QPW - 16, 16)][15] + cnt_v[pl.ds(QPW - 16, 16)][15]
    nblk = (ne + 127) // 128

    def _block(bidx, _):
        ebase = bidx * 128
        kcp = pltpu.async_copy(
            kflat.at[pl.ds((wid * CAP + ebase) * PROJ, 128 * PROJ)], kbuf,
            ksem)
        fcp = pltpu.async_copy(f2d.at[en_v.at[pl.ds(ebase, 128)]], fbuf,
                               fsem)
        kcp.wait()
        fcp.wait()
        nedge = jnp.minimum(ne - ebase, 128)

        def _edge(e, _):
            qloc = eq_v[pl.ds(ebase + e, 16)][0]
            abase = qloc * PROJ
            for g in range(PROJ // 16):
                kv = kbuf[pl.ds(e * PROJ + g * 16, 16)]
                fv = fbuf[e, pl.ds(g * 16, 16)]
                plsc.addupdate(acc.at[pl.ds(abase + g * 16, 16)], kv * fv)
            return 0

        lax.fori_loop(0, nedge, _edge, 0)
        return 0

    lax.fori_loop(0, nblk, _block, 0)

    def _div(qloc, _):
        c = jnp.maximum(cnt_v[pl.ds(qloc, 16)][0].astype(jnp.float32), 1.0)
        cs = jnp.full((16,), c, jnp.float32)
        for g in range(PROJ // 16):
            o = qloc * PROJ + g * 16
            acc[pl.ds(o, 16)] = acc[pl.ds(o, 16)] / cs
        return 0

    lax.fori_loop(0, QPW, _div, 0)
    pltpu.sync_copy(acc, o_hbm.at[pl.ds(wid * QPW * PROJ, QPW * PROJ)])


def _sc_mesh():
    return plsc.VectorSubcoreMesh(core_axis_name="c", subcore_axis_name="s",
                                  num_cores=2, num_subcores=16)


def kernel(x, input_geom, latent_queries, Wp, bp, W0, b0, W1, b1, W2, b2,
           W3, b3):
    ig = input_geom[0]
    n = ig.shape[0]
    lq = latent_queries[0]
    grid_shape = lq.shape[:-1]
    lqf = lq.reshape(-1, 3)

    xp = jnp.zeros((N_PAD, x.shape[-1]), jnp.float32).at[:n].set(x[0])
    igp = jnp.full((N_PAD, 3), 1e3, jnp.float32).at[:n].set(ig)

    f = pl.pallas_call(
        _proj_body,
        grid=(N_PAD // 2048,),
        in_specs=[
            pl.BlockSpec((2048, x.shape[-1]), lambda i: (i, 0)),
            pl.BlockSpec((x.shape[-1], PROJ), lambda i: (0, 0)),
            pl.BlockSpec((1, PROJ), lambda i: (0, 0)),
        ],
        out_specs=pl.BlockSpec((2048, PROJ), lambda i: (i, 0)),
        out_shape=jax.ShapeDtypeStruct((N_PAD, PROJ), jnp.float32),
    )(xp, Wp, bp.reshape(1, -1))

    ig2 = jnp.sum(igp * igp, axis=1).reshape(1, N_PAD)
    lq2 = jnp.sum(lqf * lqf, axis=1).reshape(Q, 1)

    narr = jnp.arange(N_PAD, dtype=jnp.int32)
    warr = jnp.arange(NW, dtype=jnp.int32)
    same_word = narr[:, None] // 32 == warr[None, :]
    bit = narr % 32
    pw = (1 << jnp.where(bit < 16, bit, bit - 16)).astype(jnp.float32)
    plo = jnp.where(same_word & (bit < 16)[:, None], pw[:, None], 0.0)
    phi = jnp.where(same_word & (bit >= 16)[:, None], pw[:, None], 0.0)

    words = pl.pallas_call(
        _mask_body,
        grid=(Q // QB,),
        in_specs=[
            pl.BlockSpec((QB, 3), lambda i: (i, 0)),
            pl.BlockSpec((3, N_PAD), lambda i: (0, 0)),
            pl.BlockSpec((QB, 1), lambda i: (i, 0)),
            pl.BlockSpec((1, N_PAD), lambda i: (0, 0)),
            pl.BlockSpec((N_PAD, NW), lambda i: (0, 0)),
            pl.BlockSpec((N_PAD, NW), lambda i: (0, 0)),
        ],
        out_specs=pl.BlockSpec((QB, NW), lambda i: (i, 0)),
        out_shape=jax.ShapeDtypeStruct((Q, NW), jnp.int32),
    )(lqf, igp.T, lq2, ig2, plo, phi)

    search = pl.kernel(
        _search_body,
        compiler_params=pltpu.CompilerParams(needs_layout_passes=False),
        out_type=(
            jax.ShapeDtypeStruct((E * 8,), jnp.float32),
            jax.ShapeDtypeStruct((E,), jnp.int32),
            jax.ShapeDtypeStruct((E,), jnp.int32),
            jax.ShapeDtypeStruct((Q,), jnp.int32),
            jax.ShapeDtypeStruct((Q,), jnp.int32),
        ),
        mesh=_sc_mesh(),
        scratch_types=[
            pltpu.VMEM((N_PAD,), jnp.float32),
            pltpu.VMEM((N_PAD,), jnp.float32),
            pltpu.VMEM((N_PAD,), jnp.float32),
            pltpu.VMEM((QPW,), jnp.float32),
            pltpu.VMEM((QPW,), jnp.float32),
            pltpu.VMEM((QPW,), jnp.float32),
            pltpu.VMEM((QPW * NW,), jnp.int32),
            pltpu.VMEM((NW + 64,), jnp.int32),
            pltpu.VMEM((CAP * 8,), jnp.float32),
            pltpu.VMEM((CAP,), jnp.int32),
            pltpu.VMEM((CAP,), jnp.int32),
            pltpu.VMEM((QPW,), jnp.int32),
            pltpu.VMEM((QPW,), jnp.int32),
        ],
    )
    kin, en, eq, cnt, start = search(
        igp[:, 0], igp[:, 1], igp[:, 2],
        lqf[:, 0], lqf[:, 1], lqf[:, 2],
        words.reshape(Q * NW))

    k = pl.pallas_call(
        _mlp_body,
        grid=(E // MLP_BLK,),
        in_specs=[
            pl.BlockSpec((MLP_BLK, 8), lambda i: (i, 0)),
            pl.BlockSpec((8, 80), lambda i: (0, 0)),
            pl.BlockSpec((1, 80), lambda i: (0, 0)),
            pl.BlockSpec((80, 80), lambda i: (0, 0)),
            pl.BlockSpec((1, 80), lambda i: (0, 0)),
            pl.BlockSpec((80, 80), lambda i: (0, 0)),
            pl.BlockSpec((1, 80), lambda i: (0, 0)),
            pl.BlockSpec((80, PROJ), lambda i: (0, 0)),
            pl.BlockSpec((1, PROJ), lambda i: (0, 0)),
        ],
        out_specs=pl.BlockSpec((MLP_BLK, PROJ), lambda i: (i, 0)),
        out_shape=jax.ShapeDtypeStruct((E, PROJ), jnp.float32),
    )(kin.reshape(E, 8),
      jnp.zeros((8, 80), jnp.float32).at[:6].set(W0),
      b0.reshape(1, -1), W1, b1.reshape(1, -1), W2, b2.reshape(1, -1),
      W3, b3.reshape(1, -1))

    agg = pl.kernel(
        _agg_body,
        out_type=jax.ShapeDtypeStruct((Q * PROJ,), jnp.float32),
        mesh=_sc_mesh(),
        scratch_types=[
            pltpu.VMEM((CAP,), jnp.int32),
            pltpu.VMEM((CAP + 16,), jnp.int32),
            pltpu.VMEM((QPW + 128,), jnp.int32),
            pltpu.VMEM((QPW + 128,), jnp.int32),
            pltpu.VMEM((128 * PROJ,), jnp.float32),
            pltpu.VMEM((128, PROJ), jnp.float32),
            pltpu.VMEM((QPW * PROJ,), jnp.float32),
            pltpu.SemaphoreType.DMA,
            pltpu.SemaphoreType.DMA,
        ],
    )
    o = agg(k.reshape(E * PROJ), f, en, eq, cnt, start)

    return o.reshape((1,) + grid_shape + (PROJ,))
